# baseline (speedup 1.0000x reference)
import jax
import jax.numpy as jnp
from jax.experimental import pallas as pl
from jax.experimental.pallas import tpu as pltpu

_NPG = 661
_NG = 4
_N = _NPG * _NG
_D = 16
_BM = 512
_GRID = -(-_N // _BM)
_NPAD = _GRID * _BM


def _phase_a(ef_ref, efp_ref, pv_ref, g_ref, b_ref, k1_ref, k2_ref, s_ref):
    ef = ef_ref[...]
    gm = jnp.dot(g_ref[...], ef, preferred_element_type=jnp.float32)
    bm = jnp.dot(b_ref[...], ef, preferred_element_type=jnp.float32)
    k1m = jnp.dot(k1_ref[...], ef, preferred_element_type=jnp.float32)
    k2m = jnp.dot(k2_ref[...], ef, preferred_element_type=jnp.float32)
    eG, fG = gm[:, :_D], gm[:, _D:]
    eB, fB = bm[:, :_D], bm[:, _D:]
    e1, f1 = k1m[:, :_D], k1m[:, _D:]
    e2, f2 = k2m[:, :_D], k2m[:, _D:]

    ev = efp_ref[:, 0:_D]
    fv = efp_ref[:, _D:2 * _D]
    pd = pv_ref[:, 0:1]
    qd = pv_ref[:, 1:2]
    gd = pv_ref[:, 2:3]
    bdg = pv_ref[:, 3:4]

    s = ev * ev + fv * fv
    base = ev * ev + fv * fv + 0.1
    alpha = pd * ev / base + qd * fv / base - eG - fB
    beta = qd * ev / base - pd * fv / base + fG + eB
    base_gb = gd * gd + bdg * bdg
    e3 = alpha * gd / base_gb + beta * bdg / base_gb
    f3 = beta * gd / base_gb - alpha * bdg / base_gb
    base1 = eG - fB
    base2 = fG + eB
    c1 = pd - s * gd
    c2 = qd + s * bdg
    new_e = (c1 * base1 + c2 * base2) / base_gb
    new_f = (c1 * base2 - c2 * base1) / base_gb

    s_ref[...] = jnp.concatenate(
        (e3, new_e, e1, e2, f3, new_f, f1, f2), axis=1)


def _phase_b(s_ref, efp_ref, wblk_ref, bvec_ref, rep_ref, bdsum_ref,
             w5ef_ref, bcat_ref, out_ref):
    sv = s_ref[...]
    idx = jax.lax.broadcasted_iota(jnp.int32, (_NPAD, 1), 0)
    parts = []
    for g in range(_NG):
        mask = jnp.logical_and(idx >= g * _NPG, idx < (g + 1) * _NPG)
        parts.append(jnp.sum(jnp.where(mask, sv, 0.0), axis=0,
                             keepdims=True))
    pooled = jnp.concatenate(parts, axis=0) / _NPG
    logits = jnp.dot(pooled, wblk_ref[...],
                     preferred_element_type=jnp.float32) + bvec_ref[...]
    a = jax.nn.sigmoid(logits)
    ae = a[:, :4]
    af = a[:, 4:]
    ae = ae / (jnp.sum(ae, axis=1, keepdims=True) + 0.0001)
    af = af / (jnp.sum(af, axis=1, keepdims=True) + 0.0001)
    a = jnp.concatenate((ae, af), axis=1)
    wrow = jnp.zeros((_NPAD, 8), dtype=jnp.float32)
    for g in range(_NG):
        mask = jnp.logical_and(idx >= g * _NPG, idx < (g + 1) * _NPG)
        wrow = wrow + jnp.where(mask, a[g:g + 1, :], 0.0)
    wexp = jnp.dot(wrow, rep_ref[...],
                   preferred_element_type=jnp.float32,
                   precision=jax.lax.Precision.HIGHEST)
    p = jnp.dot(sv * wexp, bdsum_ref[...],
                preferred_element_type=jnp.float32)
    q = jnp.dot(efp_ref[...], w5ef_ref[...],
                preferred_element_type=jnp.float32)
    out_ref[...] = jnp.tanh(p + q + bcat_ref[...])


@jax.jit
def kernel(e, f, k1, k2, G_ndiag, B_ndiag, G_diag, B_diag, Pd, Qd,
           W_v1, b_v1, W_v2, b_v2, W_ae, b_ae, W_af, b_af):
    pad = _NPAD - _N
    ef = jnp.concatenate((e, f), axis=1)
    ef_pad = jnp.pad(ef, ((0, pad), (0, 0)))
    pv = jnp.concatenate((Pd, Qd, G_diag, B_diag), axis=1)
    pv = jnp.pad(pv, ((0, pad), (0, 0)), constant_values=1.0)

    zeros16 = jnp.zeros((_D,), jnp.float32)
    cols = []
    for j in range(8):
        w = W_ae[0] if j < 4 else W_af[0]
        col = [zeros16] * 8
        col[j] = w
        cols.append(jnp.concatenate(col))
    wblk = jnp.stack(cols, axis=1)
    bvec = jnp.concatenate(
        (jnp.broadcast_to(b_ae, (4,)), jnp.broadcast_to(b_af, (4,))))
    bvec = bvec.reshape(1, 8)

    rep = jnp.repeat(jnp.eye(8, dtype=jnp.float32), _D, axis=1)

    bdsum = jnp.zeros((128, 2 * _D), jnp.float32)
    for j in range(4):
        bdsum = bdsum.at[_D * j:_D * (j + 1), 0:_D].set(
            W_v1[:, _D * j:_D * (j + 1)].T)
        bdsum = bdsum.at[64 + _D * j:64 + _D * (j + 1), _D:2 * _D].set(
            W_v2[:, _D * j:_D * (j + 1)].T)
    w5ef = jnp.zeros((2 * _D, 2 * _D), jnp.float32)
    w5ef = w5ef.at[0:_D, 0:_D].set(W_v1[:, 4 * _D:5 * _D].T)
    w5ef = w5ef.at[_D:2 * _D, _D:2 * _D].set(W_v2[:, 4 * _D:5 * _D].T)
    bcat = jnp.concatenate((b_v1, b_v2)).reshape(1, 2 * _D)

    full = lambda shape: pl.BlockSpec(shape, lambda i: (0, 0))
    row_blk = lambda w: pl.BlockSpec((_BM, w), lambda i: (i, 0))

    s_packed = pl.pallas_call(
        _phase_a,
        grid=(_GRID,),
        in_specs=[
            full((_N, 32)),
            row_blk(32),
            row_blk(4),
            pl.BlockSpec((_BM, _N), lambda i: (i, 0)),
            pl.BlockSpec((_BM, _N), lambda i: (i, 0)),
            pl.BlockSpec((_BM, _N), lambda i: (i, 0)),
            pl.BlockSpec((_BM, _N), lambda i: (i, 0)),
        ],
        out_specs=row_blk(128),
        out_shape=jax.ShapeDtypeStruct((_NPAD, 128), jnp.float32),
        compiler_params=pltpu.CompilerParams(
            dimension_semantics=("parallel",)),
    )(ef, ef_pad, pv, G_ndiag, B_ndiag, k1, k2)

    out = pl.pallas_call(
        _phase_b,
        in_specs=[
            pl.BlockSpec((_NPAD, 128), lambda: (0, 0)),
            pl.BlockSpec((_NPAD, 32), lambda: (0, 0)),
            pl.BlockSpec((128, 8), lambda: (0, 0)),
            pl.BlockSpec((1, 8), lambda: (0, 0)),
            pl.BlockSpec((8, 128), lambda: (0, 0)),
            pl.BlockSpec((128, 2 * _D), lambda: (0, 0)),
            pl.BlockSpec((2 * _D, 2 * _D), lambda: (0, 0)),
            pl.BlockSpec((1, 2 * _D), lambda: (0, 0)),
        ],
        out_specs=pl.BlockSpec((_NPAD, 2 * _D), lambda: (0, 0)),
        out_shape=jax.ShapeDtypeStruct((_NPAD, 2 * _D), jnp.float32),
    )(s_packed, ef_pad, wblk, bvec, rep, bdsum, w5ef, bcat)

    return (out[:_N, 0:_D], out[:_N, _D:2 * _D])


# speedup vs baseline: 1.0402x; 1.0402x over previous
import jax
import jax.numpy as jnp
from jax.experimental import pallas as pl
from jax.experimental.pallas import tpu as pltpu

_NPG = 661
_NG = 4
_N = _NPG * _NG
_D = 16
_BM = 128
_GRID = -(-_N // _BM)
_NPAD = _GRID * _BM


def _phase_a(ef_ref, efp_ref, pv_ref, g_ref, b_ref, k1_ref, k2_ref, s_ref):
    ef = ef_ref[...]
    gm = jnp.dot(g_ref[...], ef, preferred_element_type=jnp.float32)
    bm = jnp.dot(b_ref[...], ef, preferred_element_type=jnp.float32)
    k1m = jnp.dot(k1_ref[...], ef, preferred_element_type=jnp.float32)
    k2m = jnp.dot(k2_ref[...], ef, preferred_element_type=jnp.float32)
    eG, fG = gm[:, :_D], gm[:, _D:]
    eB, fB = bm[:, :_D], bm[:, _D:]
    e1, f1 = k1m[:, :_D], k1m[:, _D:]
    e2, f2 = k2m[:, :_D], k2m[:, _D:]

    ev = efp_ref[:, 0:_D]
    fv = efp_ref[:, _D:2 * _D]
    pd = pv_ref[:, 0:1]
    qd = pv_ref[:, 1:2]
    gd = pv_ref[:, 2:3]
    bdg = pv_ref[:, 3:4]

    s = ev * ev + fv * fv
    base = ev * ev + fv * fv + 0.1
    alpha = pd * ev / base + qd * fv / base - eG - fB
    beta = qd * ev / base - pd * fv / base + fG + eB
    base_gb = gd * gd + bdg * bdg
    e3 = alpha * gd / base_gb + beta * bdg / base_gb
    f3 = beta * gd / base_gb - alpha * bdg / base_gb
    base1 = eG - fB
    base2 = fG + eB
    c1 = pd - s * gd
    c2 = qd + s * bdg
    new_e = (c1 * base1 + c2 * base2) / base_gb
    new_f = (c1 * base2 - c2 * base1) / base_gb

    s_ref[...] = jnp.concatenate(
        (e3, new_e, e1, e2, f3, new_f, f1, f2), axis=1)


def _phase_b(s_ref, efp_ref, wblk_ref, bvec_ref, rep_ref, bdsum_ref,
             w5ef_ref, bcat_ref, out_ref):
    sv = s_ref[...]
    idx = jax.lax.broadcasted_iota(jnp.int32, (_NPAD, 1), 0)
    parts = []
    for g in range(_NG):
        mask = jnp.logical_and(idx >= g * _NPG, idx < (g + 1) * _NPG)
        parts.append(jnp.sum(jnp.where(mask, sv, 0.0), axis=0,
                             keepdims=True))
    pooled = jnp.concatenate(parts, axis=0) / _NPG
    logits = jnp.dot(pooled, wblk_ref[...],
                     preferred_element_type=jnp.float32) + bvec_ref[...]
    a = jax.nn.sigmoid(logits)
    ae = a[:, :4]
    af = a[:, 4:]
    ae = ae / (jnp.sum(ae, axis=1, keepdims=True) + 0.0001)
    af = af / (jnp.sum(af, axis=1, keepdims=True) + 0.0001)
    a = jnp.concatenate((ae, af), axis=1)
    wrow = jnp.zeros((_NPAD, 8), dtype=jnp.float32)
    for g in range(_NG):
        mask = jnp.logical_and(idx >= g * _NPG, idx < (g + 1) * _NPG)
        wrow = wrow + jnp.where(mask, a[g:g + 1, :], 0.0)
    wexp = jnp.dot(wrow, rep_ref[...],
                   preferred_element_type=jnp.float32,
                   precision=jax.lax.Precision.HIGHEST)
    p = jnp.dot(sv * wexp, bdsum_ref[...],
                preferred_element_type=jnp.float32)
    q = jnp.dot(efp_ref[...], w5ef_ref[...],
                preferred_element_type=jnp.float32)
    out_ref[...] = jnp.tanh(p + q + bcat_ref[...])


@jax.jit
def kernel(e, f, k1, k2, G_ndiag, B_ndiag, G_diag, B_diag, Pd, Qd,
           W_v1, b_v1, W_v2, b_v2, W_ae, b_ae, W_af, b_af):
    pad = _NPAD - _N
    ef = jnp.concatenate((e, f), axis=1)
    ef_pad = jnp.pad(ef, ((0, pad), (0, 0)))
    pv = jnp.concatenate((Pd, Qd, G_diag, B_diag), axis=1)
    pv = jnp.pad(pv, ((0, pad), (0, 0)), constant_values=1.0)

    zeros16 = jnp.zeros((_D,), jnp.float32)
    cols = []
    for j in range(8):
        w = W_ae[0] if j < 4 else W_af[0]
        col = [zeros16] * 8
        col[j] = w
        cols.append(jnp.concatenate(col))
    wblk = jnp.stack(cols, axis=1)
    bvec = jnp.concatenate(
        (jnp.broadcast_to(b_ae, (4,)), jnp.broadcast_to(b_af, (4,))))
    bvec = bvec.reshape(1, 8)

    rep = jnp.repeat(jnp.eye(8, dtype=jnp.float32), _D, axis=1)

    bdsum = jnp.zeros((128, 2 * _D), jnp.float32)
    for j in range(4):
        bdsum = bdsum.at[_D * j:_D * (j + 1), 0:_D].set(
            W_v1[:, _D * j:_D * (j + 1)].T)
        bdsum = bdsum.at[64 + _D * j:64 + _D * (j + 1), _D:2 * _D].set(
            W_v2[:, _D * j:_D * (j + 1)].T)
    w5ef = jnp.zeros((2 * _D, 2 * _D), jnp.float32)
    w5ef = w5ef.at[0:_D, 0:_D].set(W_v1[:, 4 * _D:5 * _D].T)
    w5ef = w5ef.at[_D:2 * _D, _D:2 * _D].set(W_v2[:, 4 * _D:5 * _D].T)
    bcat = jnp.concatenate((b_v1, b_v2)).reshape(1, 2 * _D)

    full = lambda shape: pl.BlockSpec(shape, lambda i: (0, 0))
    row_blk = lambda w: pl.BlockSpec((_BM, w), lambda i: (i, 0))

    s_packed = pl.pallas_call(
        _phase_a,
        grid=(_GRID,),
        in_specs=[
            full((_N, 32)),
            row_blk(32),
            row_blk(4),
            pl.BlockSpec((_BM, _N), lambda i: (i, 0)),
            pl.BlockSpec((_BM, _N), lambda i: (i, 0)),
            pl.BlockSpec((_BM, _N), lambda i: (i, 0)),
            pl.BlockSpec((_BM, _N), lambda i: (i, 0)),
        ],
        out_specs=row_blk(128),
        out_shape=jax.ShapeDtypeStruct((_NPAD, 128), jnp.float32),
        compiler_params=pltpu.CompilerParams(
            dimension_semantics=("parallel",)),
    )(ef, ef_pad, pv, G_ndiag, B_ndiag, k1, k2)

    out = pl.pallas_call(
        _phase_b,
        in_specs=[
            pl.BlockSpec((_NPAD, 128), lambda: (0, 0)),
            pl.BlockSpec((_NPAD, 32), lambda: (0, 0)),
            pl.BlockSpec((128, 8), lambda: (0, 0)),
            pl.BlockSpec((1, 8), lambda: (0, 0)),
            pl.BlockSpec((8, 128), lambda: (0, 0)),
            pl.BlockSpec((128, 2 * _D), lambda: (0, 0)),
            pl.BlockSpec((2 * _D, 2 * _D), lambda: (0, 0)),
            pl.BlockSpec((1, 2 * _D), lambda: (0, 0)),
        ],
        out_specs=pl.BlockSpec((_NPAD, 2 * _D), lambda: (0, 0)),
        out_shape=jax.ShapeDtypeStruct((_NPAD, 2 * _D), jnp.float32),
    )(s_packed, ef_pad, wblk, bvec, rep, bdsum, w5ef, bcat)

    return (out[:_N, 0:_D], out[:_N, _D:2 * _D])


# speedup vs baseline: 1.0797x; 1.0380x over previous
import jax
import jax.numpy as jnp
from jax.experimental import pallas as pl
from jax.experimental.pallas import tpu as pltpu

_NPG = 661
_NG = 4
_N = _NPG * _NG
_D = 16
_BM = 192
_GRID = -(-_N // _BM)
_NPAD = _GRID * _BM


def _phase_a(ef_ref, efp_ref, pv_ref, g_ref, b_ref, k1_ref, k2_ref, s_ref):
    ef = ef_ref[...]
    gm = jnp.dot(g_ref[...], ef, preferred_element_type=jnp.float32)
    bm = jnp.dot(b_ref[...], ef, preferred_element_type=jnp.float32)
    k1m = jnp.dot(k1_ref[...], ef, preferred_element_type=jnp.float32)
    k2m = jnp.dot(k2_ref[...], ef, preferred_element_type=jnp.float32)
    eG, fG = gm[:, :_D], gm[:, _D:]
    eB, fB = bm[:, :_D], bm[:, _D:]
    e1, f1 = k1m[:, :_D], k1m[:, _D:]
    e2, f2 = k2m[:, :_D], k2m[:, _D:]

    ev = efp_ref[:, 0:_D]
    fv = efp_ref[:, _D:2 * _D]
    pd = pv_ref[:, 0:1]
    qd = pv_ref[:, 1:2]
    gd = pv_ref[:, 2:3]
    bdg = pv_ref[:, 3:4]

    s = ev * ev + fv * fv
    base = ev * ev + fv * fv + 0.1
    alpha = pd * ev / base + qd * fv / base - eG - fB
    beta = qd * ev / base - pd * fv / base + fG + eB
    base_gb = gd * gd + bdg * bdg
    e3 = alpha * gd / base_gb + beta * bdg / base_gb
    f3 = beta * gd / base_gb - alpha * bdg / base_gb
    base1 = eG - fB
    base2 = fG + eB
    c1 = pd - s * gd
    c2 = qd + s * bdg
    new_e = (c1 * base1 + c2 * base2) / base_gb
    new_f = (c1 * base2 - c2 * base1) / base_gb

    s_ref[...] = jnp.concatenate(
        (e3, new_e, e1, e2, f3, new_f, f1, f2), axis=1)


def _phase_b(s_ref, efp_ref, wblk_ref, bvec_ref, rep_ref, bdsum_ref,
             w5ef_ref, bcat_ref, out_ref):
    sv = s_ref[...]
    idx = jax.lax.broadcasted_iota(jnp.int32, (_NPAD, 1), 0)
    parts = []
    for g in range(_NG):
        mask = jnp.logical_and(idx >= g * _NPG, idx < (g + 1) * _NPG)
        parts.append(jnp.sum(jnp.where(mask, sv, 0.0), axis=0,
                             keepdims=True))
    pooled = jnp.concatenate(parts, axis=0) / _NPG
    logits = jnp.dot(pooled, wblk_ref[...],
                     preferred_element_type=jnp.float32) + bvec_ref[...]
    a = jax.nn.sigmoid(logits)
    ae = a[:, :4]
    af = a[:, 4:]
    ae = ae / (jnp.sum(ae, axis=1, keepdims=True) + 0.0001)
    af = af / (jnp.sum(af, axis=1, keepdims=True) + 0.0001)
    a = jnp.concatenate((ae, af), axis=1)
    wrow = jnp.zeros((_NPAD, 8), dtype=jnp.float32)
    for g in range(_NG):
        mask = jnp.logical_and(idx >= g * _NPG, idx < (g + 1) * _NPG)
        wrow = wrow + jnp.where(mask, a[g:g + 1, :], 0.0)
    wexp = jnp.dot(wrow, rep_ref[...],
                   preferred_element_type=jnp.float32,
                   precision=jax.lax.Precision.HIGHEST)
    p = jnp.dot(sv * wexp, bdsum_ref[...],
                preferred_element_type=jnp.float32)
    q = jnp.dot(efp_ref[...], w5ef_ref[...],
                preferred_element_type=jnp.float32)
    out_ref[...] = jnp.tanh(p + q + bcat_ref[...])


@jax.jit
def kernel(e, f, k1, k2, G_ndiag, B_ndiag, G_diag, B_diag, Pd, Qd,
           W_v1, b_v1, W_v2, b_v2, W_ae, b_ae, W_af, b_af):
    pad = _NPAD - _N
    ef = jnp.concatenate((e, f), axis=1)
    ef_pad = jnp.pad(ef, ((0, pad), (0, 0)))
    pv = jnp.concatenate((Pd, Qd, G_diag, B_diag), axis=1)
    pv = jnp.pad(pv, ((0, pad), (0, 0)), constant_values=1.0)

    zeros16 = jnp.zeros((_D,), jnp.float32)
    cols = []
    for j in range(8):
        w = W_ae[0] if j < 4 else W_af[0]
        col = [zeros16] * 8
        col[j] = w
        cols.append(jnp.concatenate(col))
    wblk = jnp.stack(cols, axis=1)
    bvec = jnp.concatenate(
        (jnp.broadcast_to(b_ae, (4,)), jnp.broadcast_to(b_af, (4,))))
    bvec = bvec.reshape(1, 8)

    rep = jnp.repeat(jnp.eye(8, dtype=jnp.float32), _D, axis=1)

    bdsum = jnp.zeros((128, 2 * _D), jnp.float32)
    for j in range(4):
        bdsum = bdsum.at[_D * j:_D * (j + 1), 0:_D].set(
            W_v1[:, _D * j:_D * (j + 1)].T)
        bdsum = bdsum.at[64 + _D * j:64 + _D * (j + 1), _D:2 * _D].set(
            W_v2[:, _D * j:_D * (j + 1)].T)
    w5ef = jnp.zeros((2 * _D, 2 * _D), jnp.float32)
    w5ef = w5ef.at[0:_D, 0:_D].set(W_v1[:, 4 * _D:5 * _D].T)
    w5ef = w5ef.at[_D:2 * _D, _D:2 * _D].set(W_v2[:, 4 * _D:5 * _D].T)
    bcat = jnp.concatenate((b_v1, b_v2)).reshape(1, 2 * _D)

    full = lambda shape: pl.BlockSpec(shape, lambda i: (0, 0))
    row_blk = lambda w: pl.BlockSpec((_BM, w), lambda i: (i, 0))

    s_packed = pl.pallas_call(
        _phase_a,
        grid=(_GRID,),
        in_specs=[
            full((_N, 32)),
            row_blk(32),
            row_blk(4),
            pl.BlockSpec((_BM, _N), lambda i: (i, 0)),
            pl.BlockSpec((_BM, _N), lambda i: (i, 0)),
            pl.BlockSpec((_BM, _N), lambda i: (i, 0)),
            pl.BlockSpec((_BM, _N), lambda i: (i, 0)),
        ],
        out_specs=row_blk(128),
        out_shape=jax.ShapeDtypeStruct((_NPAD, 128), jnp.float32),
        compiler_params=pltpu.CompilerParams(
            dimension_semantics=("parallel",)),
    )(ef, ef_pad, pv, G_ndiag, B_ndiag, k1, k2)

    out = pl.pallas_call(
        _phase_b,
        in_specs=[
            pl.BlockSpec((_NPAD, 128), lambda: (0, 0)),
            pl.BlockSpec((_NPAD, 32), lambda: (0, 0)),
            pl.BlockSpec((128, 8), lambda: (0, 0)),
            pl.BlockSpec((1, 8), lambda: (0, 0)),
            pl.BlockSpec((8, 128), lambda: (0, 0)),
            pl.BlockSpec((128, 2 * _D), lambda: (0, 0)),
            pl.BlockSpec((2 * _D, 2 * _D), lambda: (0, 0)),
            pl.BlockSpec((1, 2 * _D), lambda: (0, 0)),
        ],
        out_specs=pl.BlockSpec((_NPAD, 2 * _D), lambda: (0, 0)),
        out_shape=jax.ShapeDtypeStruct((_NPAD, 2 * _D), jnp.float32),
    )(s_packed, ef_pad, wblk, bvec, rep, bdsum, w5ef, bcat)

    return (out[:_N, 0:_D], out[:_N, _D:2 * _D])


# speedup vs baseline: 1.0916x; 1.0110x over previous
import jax
import jax.numpy as jnp
from jax.experimental import pallas as pl
from jax.experimental.pallas import tpu as pltpu

_NPG = 661
_NG = 4
_N = _NPG * _NG
_D = 16
_BM = 224
_GRID = -(-_N // _BM)
_NPAD = _GRID * _BM


def _phase_a(ef_ref, efp_ref, pv_ref, g_ref, b_ref, k1_ref, k2_ref, s_ref):
    ef = ef_ref[...]
    gm = jnp.dot(g_ref[...], ef, preferred_element_type=jnp.float32)
    bm = jnp.dot(b_ref[...], ef, preferred_element_type=jnp.float32)
    k1m = jnp.dot(k1_ref[...], ef, preferred_element_type=jnp.float32)
    k2m = jnp.dot(k2_ref[...], ef, preferred_element_type=jnp.float32)
    eG, fG = gm[:, :_D], gm[:, _D:]
    eB, fB = bm[:, :_D], bm[:, _D:]
    e1, f1 = k1m[:, :_D], k1m[:, _D:]
    e2, f2 = k2m[:, :_D], k2m[:, _D:]

    ev = efp_ref[:, 0:_D]
    fv = efp_ref[:, _D:2 * _D]
    pd = pv_ref[:, 0:1]
    qd = pv_ref[:, 1:2]
    gd = pv_ref[:, 2:3]
    bdg = pv_ref[:, 3:4]

    s = ev * ev + fv * fv
    base = ev * ev + fv * fv + 0.1
    alpha = pd * ev / base + qd * fv / base - eG - fB
    beta = qd * ev / base - pd * fv / base + fG + eB
    base_gb = gd * gd + bdg * bdg
    e3 = alpha * gd / base_gb + beta * bdg / base_gb
    f3 = beta * gd / base_gb - alpha * bdg / base_gb
    base1 = eG - fB
    base2 = fG + eB
    c1 = pd - s * gd
    c2 = qd + s * bdg
    new_e = (c1 * base1 + c2 * base2) / base_gb
    new_f = (c1 * base2 - c2 * base1) / base_gb

    s_ref[...] = jnp.concatenate(
        (e3, new_e, e1, e2, f3, new_f, f1, f2), axis=1)


def _phase_b(s_ref, efp_ref, wblk_ref, bvec_ref, rep_ref, bdsum_ref,
             w5ef_ref, bcat_ref, out_ref):
    sv = s_ref[...]
    idx = jax.lax.broadcasted_iota(jnp.int32, (_NPAD, 1), 0)
    parts = []
    for g in range(_NG):
        mask = jnp.logical_and(idx >= g * _NPG, idx < (g + 1) * _NPG)
        parts.append(jnp.sum(jnp.where(mask, sv, 0.0), axis=0,
                             keepdims=True))
    pooled = jnp.concatenate(parts, axis=0) / _NPG
    logits = jnp.dot(pooled, wblk_ref[...],
                     preferred_element_type=jnp.float32) + bvec_ref[...]
    a = jax.nn.sigmoid(logits)
    ae = a[:, :4]
    af = a[:, 4:]
    ae = ae / (jnp.sum(ae, axis=1, keepdims=True) + 0.0001)
    af = af / (jnp.sum(af, axis=1, keepdims=True) + 0.0001)
    a = jnp.concatenate((ae, af), axis=1)
    wrow = jnp.zeros((_NPAD, 8), dtype=jnp.float32)
    for g in range(_NG):
        mask = jnp.logical_and(idx >= g * _NPG, idx < (g + 1) * _NPG)
        wrow = wrow + jnp.where(mask, a[g:g + 1, :], 0.0)
    wexp = jnp.dot(wrow, rep_ref[...],
                   preferred_element_type=jnp.float32,
                   precision=jax.lax.Precision.HIGHEST)
    p = jnp.dot(sv * wexp, bdsum_ref[...],
                preferred_element_type=jnp.float32)
    q = jnp.dot(efp_ref[...], w5ef_ref[...],
                preferred_element_type=jnp.float32)
    out_ref[...] = jnp.tanh(p + q + bcat_ref[...])


@jax.jit
def kernel(e, f, k1, k2, G_ndiag, B_ndiag, G_diag, B_diag, Pd, Qd,
           W_v1, b_v1, W_v2, b_v2, W_ae, b_ae, W_af, b_af):
    pad = _NPAD - _N
    ef = jnp.concatenate((e, f), axis=1)
    ef_pad = jnp.pad(ef, ((0, pad), (0, 0)))
    pv = jnp.concatenate((Pd, Qd, G_diag, B_diag), axis=1)
    pv = jnp.pad(pv, ((0, pad), (0, 0)), constant_values=1.0)

    zeros16 = jnp.zeros((_D,), jnp.float32)
    cols = []
    for j in range(8):
        w = W_ae[0] if j < 4 else W_af[0]
        col = [zeros16] * 8
        col[j] = w
        cols.append(jnp.concatenate(col))
    wblk = jnp.stack(cols, axis=1)
    bvec = jnp.concatenate(
        (jnp.broadcast_to(b_ae, (4,)), jnp.broadcast_to(b_af, (4,))))
    bvec = bvec.reshape(1, 8)

    rep = jnp.repeat(jnp.eye(8, dtype=jnp.float32), _D, axis=1)

    bdsum = jnp.zeros((128, 2 * _D), jnp.float32)
    for j in range(4):
        bdsum = bdsum.at[_D * j:_D * (j + 1), 0:_D].set(
            W_v1[:, _D * j:_D * (j + 1)].T)
        bdsum = bdsum.at[64 + _D * j:64 + _D * (j + 1), _D:2 * _D].set(
            W_v2[:, _D * j:_D * (j + 1)].T)
    w5ef = jnp.zeros((2 * _D, 2 * _D), jnp.float32)
    w5ef = w5ef.at[0:_D, 0:_D].set(W_v1[:, 4 * _D:5 * _D].T)
    w5ef = w5ef.at[_D:2 * _D, _D:2 * _D].set(W_v2[:, 4 * _D:5 * _D].T)
    bcat = jnp.concatenate((b_v1, b_v2)).reshape(1, 2 * _D)

    full = lambda shape: pl.BlockSpec(shape, lambda i: (0, 0))
    row_blk = lambda w: pl.BlockSpec((_BM, w), lambda i: (i, 0))

    s_packed = pl.pallas_call(
        _phase_a,
        grid=(_GRID,),
        in_specs=[
            full((_N, 32)),
            row_blk(32),
            row_blk(4),
            pl.BlockSpec((_BM, _N), lambda i: (i, 0)),
            pl.BlockSpec((_BM, _N), lambda i: (i, 0)),
            pl.BlockSpec((_BM, _N), lambda i: (i, 0)),
            pl.BlockSpec((_BM, _N), lambda i: (i, 0)),
        ],
        out_specs=row_blk(128),
        out_shape=jax.ShapeDtypeStruct((_NPAD, 128), jnp.float32),
        compiler_params=pltpu.CompilerParams(
            dimension_semantics=("parallel",)),
    )(ef, ef_pad, pv, G_ndiag, B_ndiag, k1, k2)

    out = pl.pallas_call(
        _phase_b,
        in_specs=[
            pl.BlockSpec((_NPAD, 128), lambda: (0, 0)),
            pl.BlockSpec((_NPAD, 32), lambda: (0, 0)),
            pl.BlockSpec((128, 8), lambda: (0, 0)),
            pl.BlockSpec((1, 8), lambda: (0, 0)),
            pl.BlockSpec((8, 128), lambda: (0, 0)),
            pl.BlockSpec((128, 2 * _D), lambda: (0, 0)),
            pl.BlockSpec((2 * _D, 2 * _D), lambda: (0, 0)),
            pl.BlockSpec((1, 2 * _D), lambda: (0, 0)),
        ],
        out_specs=pl.BlockSpec((_NPAD, 2 * _D), lambda: (0, 0)),
        out_shape=jax.ShapeDtypeStruct((_NPAD, 2 * _D), jnp.float32),
    )(s_packed, ef_pad, wblk, bvec, rep, bdsum, w5ef, bcat)

    return (out[:_N, 0:_D], out[:_N, _D:2 * _D])
